# Initial kernel scaffold; baseline (speedup 1.0000x reference)
import jax
import jax.numpy as jnp
from jax.experimental import pallas as pl


def kernel(token_ids, weights):
    raise NotImplementedError("write your pallas kernel here")


# speedup vs baseline: 8.1450x; 8.1450x over previous
import functools

import jax
import jax.numpy as jnp
from jax import lax
from jax.experimental import pallas as pl
from jax.experimental.pallas import tpu as pltpu
from jax.experimental.pallas import tpu_sc as plsc

D_MODEL = 128
BLOCK = 128
NK = 4


def _body(nb_per_worker, nc, idx_hbm, table_hbm, out_hbm, idx_v, rows_v, sem):
    wid = lax.axis_index("s") * nc + lax.axis_index("c")
    blk0 = wid * nb_per_worker

    @pl.loop(0, nb_per_worker // NK)
    def _chunk(i):
        b = blk0 + i * NK
        pltpu.sync_copy(idx_hbm.at[pl.ds(b, NK)], idx_v)
        copies = [
            pltpu.async_copy(table_hbm.at[idx_v.at[j]], rows_v.at[j], sem)
            for j in range(NK)
        ]
        for c in copies:
            c.wait()
        pltpu.sync_copy(rows_v, out_hbm.at[pl.ds(b, NK)])


@functools.partial(jax.jit, static_argnums=())
def kernel(token_ids, weights):
    b, t = token_ids.shape
    vocab, d = weights.shape
    total = b * t
    nb = total // BLOCK
    info = plsc.get_sparse_core_info()
    nw = info.num_cores * info.num_subcores
    nb_per_worker = nb // nw

    idx2d = token_ids.reshape(nb, BLOCK).astype(jnp.int32)
    mesh = plsc.VectorSubcoreMesh(core_axis_name="c", subcore_axis_name="s")
    run = pl.kernel(
        functools.partial(_body, nb_per_worker, info.num_cores),
        out_type=jax.ShapeDtypeStruct((nb, BLOCK, d), jnp.float32),
        mesh=mesh,
        scratch_types=[
            pltpu.VMEM((NK, BLOCK), jnp.int32),
            pltpu.VMEM((NK, BLOCK, d), jnp.float32),
            pltpu.SemaphoreType.DMA,
        ],
    )
    out = run(idx2d, weights)
    return out.reshape(b, t, d)


# speedup vs baseline: 9.1584x; 1.1244x over previous
import functools

import jax
import jax.numpy as jnp
from jax import lax
from jax.experimental import pallas as pl
from jax.experimental.pallas import tpu as pltpu
from jax.experimental.pallas import tpu_sc as plsc

D_MODEL = 128
BLOCK = 128
NK = 2


def _body(bpw, nc, idx_hbm, table_hbm, out_hbm,
          idx_all, rows_v, gsem0, gsem1, wsem0, wsem1):
    wid = lax.axis_index("s") * nc + lax.axis_index("c")
    blk0 = wid * bpw
    ch = bpw // NK
    gsems = (gsem0, gsem1)
    wsems = (wsem0, wsem1)

    pltpu.sync_copy(idx_hbm.at[pl.ds(blk0, bpw)], idx_all)

    def fire_g(c, slot):
        for j in range(NK):
            pltpu.async_copy(table_hbm.at[idx_all.at[c * NK + j]],
                             rows_v.at[slot].at[j], gsems[slot])

    def wait_g(c, slot):
        for j in range(NK):
            pltpu.make_async_copy(table_hbm.at[idx_all.at[c * NK + j]],
                                  rows_v.at[slot].at[j], gsems[slot]).wait()

    def fire_w(c, slot):
        pltpu.async_copy(rows_v.at[slot],
                         out_hbm.at[pl.ds(blk0 + c * NK, NK)], wsems[slot])

    def wait_w(slot):
        pltpu.make_async_copy(rows_v.at[slot],
                              out_hbm.at[pl.ds(blk0, NK)], wsems[slot]).wait()

    def stage(c, slot, fire_next=True, wait_prev=True):
        wait_g(c, slot)
        if wait_prev:
            wait_w(1 - slot)
        if fire_next:
            fire_g(c + 1, 1 - slot)
        fire_w(c, slot)

    fire_g(0, 0)
    stage(0, 0, wait_prev=False)
    stage(1, 1)

    @pl.loop(1, ch // 2 - 1)
    def _main(k):
        c = 2 * k
        stage(c, 0)
        stage(c + 1, 1)

    stage(ch - 2, 0)
    stage(ch - 1, 1, fire_next=False)
    wait_w(1)


def kernel(token_ids, weights):
    b, t = token_ids.shape
    vocab, d = weights.shape
    nb = (b * t) // BLOCK
    info = plsc.get_sparse_core_info()
    nw = info.num_cores * info.num_subcores
    bpw = nb // nw

    idx2d = token_ids.reshape(nb, BLOCK).astype(jnp.int32)
    mesh = plsc.VectorSubcoreMesh(core_axis_name="c", subcore_axis_name="s")
    run = pl.kernel(
        functools.partial(_body, bpw, info.num_cores),
        out_type=jax.ShapeDtypeStruct((nb, BLOCK, d), jnp.float32),
        mesh=mesh,
        scratch_types=[
            pltpu.VMEM((bpw, BLOCK), jnp.int32),
            pltpu.VMEM((2, NK, BLOCK, d), jnp.float32),
            pltpu.SemaphoreType.DMA,
            pltpu.SemaphoreType.DMA,
            pltpu.SemaphoreType.DMA,
            pltpu.SemaphoreType.DMA,
        ],
    )
    out = run(idx2d, weights)
    return out.reshape(b, t, d)


# speedup vs baseline: 9.2102x; 1.0057x over previous
import functools

import jax
import jax.numpy as jnp
from jax import lax
from jax.experimental import pallas as pl
from jax.experimental.pallas import tpu as pltpu
from jax.experimental.pallas import tpu_sc as plsc

BLOCK = 128
S = 5
L = 3


def _body(bpw, nc, idx_hbm, table_hbm, out_hbm, idx_all, rows_v, *sems):
    wid = lax.axis_index("s") * nc + lax.axis_index("c")
    blk0 = wid * bpw
    gsems, wsems = sems[:S], sems[S:]

    pltpu.sync_copy(idx_hbm.at[pl.ds(blk0, bpw)], idx_all)

    def fire_g(c, s):
        pltpu.async_copy(table_hbm.at[idx_all.at[c]], rows_v.at[s], gsems[s])

    def wait_g(c, s):
        pltpu.make_async_copy(table_hbm.at[idx_all.at[c]],
                              rows_v.at[s], gsems[s]).wait()

    def fire_w(c, s):
        pltpu.async_copy(rows_v.at[s], out_hbm.at[blk0 + c], wsems[s])

    def wait_w(s):
        pltpu.make_async_copy(rows_v.at[s], out_hbm.at[blk0], wsems[s]).wait()

    def stage(c, s, do_wait_w=True, fire_next=True):
        wait_g(c, s)
        fire_w(c, s)
        if do_wait_w or fire_next:
            t = (s + L) % S
        if do_wait_w:
            wait_w(t)
        if fire_next:
            fire_g(c + L, t)

    for c in range(L):
        fire_g(c, c)
    for c in range(S - L):
        stage(c, c, do_wait_w=False)
    for c in range(S - L, S):
        stage(c, c)

    @pl.loop(1, bpw // S - 1)
    def _main(k):
        c0 = S * k
        for s in range(S):
            stage(c0 + s, s)

    for c in range(bpw - S, bpw):
        stage(c, c % S, fire_next=(c + L < bpw))
    for c in range(bpw + L - S, bpw):
        wait_w(c % S)


def kernel(token_ids, weights):
    b, t = token_ids.shape
    vocab, d = weights.shape
    nb = (b * t) // BLOCK
    info = plsc.get_sparse_core_info()
    nw = info.num_cores * info.num_subcores
    bpw = nb // nw

    idx2d = token_ids.reshape(nb, BLOCK).astype(jnp.int32)
    mesh = plsc.VectorSubcoreMesh(core_axis_name="c", subcore_axis_name="s")
    run = pl.kernel(
        functools.partial(_body, bpw, info.num_cores),
        out_type=jax.ShapeDtypeStruct((nb, BLOCK, d), jnp.float32),
        mesh=mesh,
        scratch_types=[
            pltpu.VMEM((bpw, BLOCK), jnp.int32),
            pltpu.VMEM((S, BLOCK, d), jnp.float32),
        ] + [pltpu.SemaphoreType.DMA] * (2 * S),
    )
    out = run(idx2d, weights)
    return out.reshape(b, t, d)


# speedup vs baseline: 9.2262x; 1.0017x over previous
import functools

import jax
import jax.numpy as jnp
from jax import lax
from jax.experimental import pallas as pl
from jax.experimental.pallas import tpu as pltpu
from jax.experimental.pallas import tpu_sc as plsc

BLOCK = 128
S = 6
L = 4


def _body(bpw, nc, idx_hbm, table_hbm, out_hbm, idx_all, rows_v, *sems):
    wid = lax.axis_index("s") * nc + lax.axis_index("c")
    blk0 = wid * bpw
    gsems, wsems = sems[:S], sems[S:]

    pltpu.sync_copy(idx_hbm.at[pl.ds(blk0, bpw)], idx_all)

    def fire_g(c, s):
        pltpu.async_copy(table_hbm.at[idx_all.at[c]], rows_v.at[s], gsems[s])

    def wait_g(c, s):
        pltpu.make_async_copy(table_hbm.at[idx_all.at[c]],
                              rows_v.at[s], gsems[s]).wait()

    def fire_w(c, s):
        pltpu.async_copy(rows_v.at[s], out_hbm.at[blk0 + c], wsems[s])

    def wait_w(s):
        pltpu.make_async_copy(rows_v.at[s], out_hbm.at[blk0], wsems[s]).wait()

    def stage(c, s, do_wait_w=True, fire_next=True):
        wait_g(c, s)
        fire_w(c, s)
        if do_wait_w or fire_next:
            t = (s + L) % S
        if do_wait_w:
            wait_w(t)
        if fire_next:
            fire_g(c + L, t)

    for c in range(L):
        fire_g(c, c % S)
    for c in range(S):
        stage(c, c % S, do_wait_w=(c + L >= S))

    k1 = (bpw - L) // S

    @pl.loop(1, k1)
    def _main(k):
        c0 = S * k
        for s in range(S):
            stage(c0 + s, s)

    for c in range(S * k1, bpw):
        stage(c, c % S, fire_next=(c + L < bpw))
    for c in range(bpw - (S - L), bpw):
        wait_w(c % S)


def kernel(token_ids, weights):
    b, t = token_ids.shape
    vocab, d = weights.shape
    nb = (b * t) // BLOCK
    info = plsc.get_sparse_core_info()
    nw = info.num_cores * info.num_subcores
    bpw = nb // nw

    idx2d = token_ids.reshape(nb, BLOCK).astype(jnp.int32)
    mesh = plsc.VectorSubcoreMesh(core_axis_name="c", subcore_axis_name="s")
    run = pl.kernel(
        functools.partial(_body, bpw, info.num_cores),
        out_type=jax.ShapeDtypeStruct((nb, BLOCK, d), jnp.float32),
        mesh=mesh,
        scratch_types=[
            pltpu.VMEM((bpw, BLOCK), jnp.int32),
            pltpu.VMEM((S, BLOCK, d), jnp.float32),
        ] + [pltpu.SemaphoreType.DMA] * (2 * S),
    )
    out = run(idx2d, weights)
    return out.reshape(b, t, d)
